# baseline (speedup 1.0000x reference)
import functools

import jax
import jax.numpy as jnp
from jax import lax
from jax.experimental import pallas as pl
from jax.experimental.pallas import tpu as pltpu
from jax.experimental.pallas import tpu_sc as plsc

N = 10000
NP = 10240
E = 320000
H = 128
T = 3
L = 3

NUM_CORES = 2
NUM_SUBCORES = 16
NUM_WORKERS = NUM_CORES * NUM_SUBCORES
GC = 48
SLOTS = 6
CHUNKS = 216
EPW_P = CHUNKS * GC
EP = NUM_WORKERS * EPW_P
HALF = CHUNKS // 2
HWORDS = HALF * GC
ROWS_PER_TILE = NP // NUM_SUBCORES
FLUSH_CHUNK = 128


def _make_sc_agg(num_towers: int):
    mesh = plsc.VectorSubcoreMesh(core_axis_name="c", subcore_axis_name="s")

    def body(h_hbm, src_hbm, dst_hbm, zeros_hbm, out_hbm,
             src_h, dst_h, ring, acc, *sems):
        c = lax.axis_index("c")
        s = lax.axis_index("s")
        wid = c * NUM_SUBCORES + s
        gsems = sems[:SLOTS]
        ssems = sems[SLOTS:]

        def stage(half):
            pltpu.sync_copy(src_hbm.at[wid, half], src_h)
            pltpu.sync_copy(dst_hbm.at[wid, half], dst_h)

        def stage_src(half):
            pltpu.sync_copy(src_hbm.at[wid, half], src_h)

        def stage_dst(half):
            pltpu.sync_copy(dst_hbm.at[wid, half], dst_h)

        stage(0)

        for t in range(num_towers):
            pltpu.sync_copy(zeros_hbm, ring.at[pl.ds(0, FLUSH_CHUNK)])
            r0 = s * ROWS_PER_TILE
            for k in range(ROWS_PER_TILE // FLUSH_CHUNK):
                pltpu.sync_copy(
                    ring.at[pl.ds(0, FLUSH_CHUNK)],
                    acc.at[pl.ds(r0 + k * FLUSH_CHUNK, FLUSH_CHUNK)])
            plsc.subcore_barrier()
            if t > 0:
                stage(0)

            table = h_hbm.at[t]

            def fire_g(cchunk, q):
                off = pl.multiple_of(cchunk * GC, GC)
                pltpu.async_copy(
                    table.at[src_h.at[pl.ds(off, GC)]],
                    ring.at[pl.ds(q * GC, GC)], gsems[q])

            def wait_g(cchunk, q):
                off = pl.multiple_of(cchunk * GC, GC)
                pltpu.make_async_copy(
                    table.at[src_h.at[pl.ds(off, GC)]],
                    ring.at[pl.ds(q * GC, GC)], gsems[q]).wait()

            def fire_s(cchunk, q):
                for i in range(3):
                    off = pl.multiple_of(cchunk * GC + i * 16, 16)
                    idx = dst_h[pl.ds(off, 16)]
                    pltpu.async_copy(
                        ring.at[pl.ds(q * GC + i * 16, 16)],
                        acc.at[idx], ssems[q], add=True)

            def wait_s(cchunk, q):
                for i in range(3):
                    off = pl.multiple_of(cchunk * GC + i * 16, 16)
                    idx = dst_h[pl.ds(off, 16)]
                    pltpu.make_async_copy(
                        ring.at[pl.ds(q * GC + i * 16, 16)],
                        acc.at[idx], ssems[q]).wait()

            def slot(cc, q, wait_prev, fire_ahead):
                wait_g(cc, q)
                fire_s(cc, q)
                q4 = (q + 4) % SLOTS
                if wait_prev:
                    wait_s(cc - 2, q4)
                if fire_ahead:
                    fire_g(cc + 4, q4)

            for q in range(4):
                fire_g(q, q)
            slot(0, 0, False, True)
            slot(1, 1, False, True)

            def six_a(p, carry):
                base = 6 * p + 2
                for j in range(6):
                    slot(base + j, (2 + j) % SLOTS, True, True)
                return carry

            lax.fori_loop(0, 17, six_a, 0)

            stage_src(1)
            for cc in range(104, 108):
                wait_g(cc, cc % SLOTS)
                fire_s(cc, cc % SLOTS)
                q4 = (cc + 4) % SLOTS
                wait_s(cc - 2, q4)
                fire_g(cc + 4 - HALF, q4)

            stage_dst(1)
            for cc in range(108, 110):
                hc = cc - HALF
                wait_g(hc, cc % SLOTS)
                fire_s(hc, cc % SLOTS)
                q4 = (cc + 4) % SLOTS
                wait_s(cc - 2, q4)
                fire_g(hc + 4, q4)

            def six_b(p, carry):
                base = 6 * p + 2
                for j in range(6):
                    slot(base + j, (2 + j) % SLOTS, True, True)
                return carry

            lax.fori_loop(0, 17, six_b, 0)

            for cc in range(212, 216):
                hc = cc - HALF
                wait_g(hc, cc % SLOTS)
                fire_s(hc, cc % SLOTS)
                wait_s(hc - 2, (cc + 4) % SLOTS)
            wait_s(106, (214 % SLOTS))
            wait_s(107, (215 % SLOTS))

            plsc.subcore_barrier()

            for k in range(ROWS_PER_TILE // FLUSH_CHUNK):
                off = r0 + k * FLUSH_CHUNK
                pltpu.sync_copy(acc.at[pl.ds(off, FLUSH_CHUNK)],
                                ring.at[pl.ds(0, FLUSH_CHUNK)])
                pltpu.sync_copy(ring.at[pl.ds(0, FLUSH_CHUNK)],
                                out_hbm.at[c, t, pl.ds(off, FLUSH_CHUNK)])
            plsc.subcore_barrier()

    return pl.kernel(
        body,
        out_type=jax.ShapeDtypeStruct((NUM_CORES, num_towers, NP, H),
                                      jnp.float32),
        mesh=mesh,
        scratch_types=(
            [pltpu.VMEM((HWORDS,), jnp.int32),
             pltpu.VMEM((HWORDS,), jnp.int32),
             pltpu.VMEM((SLOTS * GC, H), jnp.float32),
             pltpu.VMEM_SHARED((NP, H), jnp.float32)]
            + [pltpu.SemaphoreType.DMA] * (2 * SLOTS)
        ),
    )


_sc_agg_1 = _make_sc_agg(1)
_sc_agg_3 = _make_sc_agg(T)

BN = 1024
GRID = NP // BN


def _mm(a, w):
    return lax.dot_general(a, w, (((1,), (0,)), ((), ())),
                           preferred_element_type=jnp.float32,
                           precision=lax.Precision.HIGHEST)


def _mlp_first_body(scale_ref, x_ref, aggp_ref, w1_ref, b1_ref, w2_ref,
                    b2_ref, out_ref):
    agg = aggp_ref[0] + aggp_ref[1]
    x = x_ref[...]
    for t in range(T):
        u = scale_ref[t] * x + agg
        v = jnp.maximum(_mm(u, w1_ref[t]) + b1_ref[t], 0.0)
        w = jnp.maximum(_mm(v, w2_ref[t]) + b2_ref[t], 0.0)
        out_ref[t] = w


def _mlp_mid_body(scale_ref, h_ref, aggp_ref, w1_ref, b1_ref, w2_ref,
                  b2_ref, out_ref):
    for t in range(T):
        u = scale_ref[t] * h_ref[t] + (aggp_ref[0, t] + aggp_ref[1, t])
        v = jnp.maximum(_mm(u, w1_ref[t]) + b1_ref[t], 0.0)
        w = jnp.maximum(_mm(v, w2_ref[t]) + b2_ref[t], 0.0)
        out_ref[t] = w


_W_SPEC = pl.BlockSpec((T, H, H), lambda i: (0, 0, 0))
_B_SPEC = pl.BlockSpec((T, H), lambda i: (0, 0))
_H3_SPEC = pl.BlockSpec((T, BN, H), lambda i: (0, i, 0))

_mlp_first = pl.pallas_call(
    _mlp_first_body,
    grid=(GRID,),
    in_specs=[
        pl.BlockSpec(memory_space=pltpu.SMEM),
        pl.BlockSpec((BN, H), lambda i: (i, 0)),
        pl.BlockSpec((NUM_CORES, BN, H), lambda i: (0, i, 0)),
        _W_SPEC, _B_SPEC, _W_SPEC, _B_SPEC,
    ],
    out_specs=_H3_SPEC,
    out_shape=jax.ShapeDtypeStruct((T, NP, H), jnp.float32),
)

_mlp_mid = pl.pallas_call(
    _mlp_mid_body,
    grid=(GRID,),
    in_specs=[
        pl.BlockSpec(memory_space=pltpu.SMEM),
        _H3_SPEC,
        pl.BlockSpec((NUM_CORES, T, BN, H), lambda i: (0, 0, i, 0)),
        _W_SPEC, _B_SPEC, _W_SPEC, _B_SPEC,
    ],
    out_specs=_H3_SPEC,
    out_shape=jax.ShapeDtypeStruct((T, NP, H), jnp.float32),
)


def kernel(x, edge_index, W1, b1, W2, b2, eps):
    src = jnp.concatenate(
        [edge_index[0], jnp.zeros((EP - E,), jnp.int32)]
    ).reshape(NUM_WORKERS, 2, HWORDS)
    dst = jnp.concatenate(
        [edge_index[1], jnp.full((EP - E,), N, jnp.int32)]
    ).reshape(NUM_WORKERS, 2, HWORDS)
    scale = 1.0 + eps
    zeros = jnp.zeros((FLUSH_CHUNK, H), jnp.float32)
    xp = jnp.pad(x, ((0, NP - N), (0, 0)))

    aggp0 = _sc_agg_1(xp[None], src, dst, zeros)
    h = _mlp_first(scale[:, 0], xp, aggp0[:, 0],
                   W1[:, 0], b1[:, 0], W2[:, 0], b2[:, 0])
    for l in range(1, L):
        aggp = _sc_agg_3(h, src, dst, zeros)
        h = _mlp_mid(scale[:, l], h, aggp,
                     W1[:, l], b1[:, l], W2[:, l], b2[:, l])
    return jnp.transpose(h[:, :N], (1, 0, 2))
